# baseline (speedup 1.0000x reference)
import jax
import jax.numpy as jnp
from jax.experimental import pallas as pl
from jax.experimental.pallas import tpu as pltpu

K_SELECT = 300
L_REAL = 1203
L_PAD = 1280
N_ROWS = 5000
NTILE = 5
TN = N_ROWS // NTILE
NEG = -1e30
BIGI = 2_000_000_000


def _fused_body(logits_ref, wt_ref, boxes_ref, scale_ref,
                scores_ref, labels_ref, nidx_ref, boxes_out_ref, pv_ref):
    col = jax.lax.broadcasted_iota(jnp.int32, (TN, L_PAD), 1)
    v_parts, fl_parts = [], []
    for t in range(NTILE):
        x = logits_ref[0, t * TN:(t + 1) * TN, :]
        s = jax.nn.sigmoid(x)
        probs = jnp.dot(s, wt_ref[...], preferred_element_type=jnp.float32)
        probs = jnp.where(col < L_REAL, probs, NEG)
        pv_ref[t * TN:(t + 1) * TN, :] = probs
        vt = jnp.max(probs, axis=1)
        v_parts.append(vt)
        fl_parts.append(jnp.min(
            jnp.where(probs == vt[:, None], col, BIGI), axis=1))
    V = jnp.stack(v_parts)
    FL = jnp.stack(fl_parts)

    pos2 = jax.lax.broadcasted_iota(jnp.int32, (NTILE, TN), 0) * TN + \
        jax.lax.broadcasted_iota(jnp.int32, (NTILE, TN), 1)
    iota_l = jax.lax.broadcasted_iota(jnp.int32, (1, L_PAD), 1)
    iota384 = jax.lax.broadcasted_iota(jnp.int32, (1, 384), 1)

    def body(t, carry):
        V, FL, gm, r, outv, outn, outl = carry
        rsel = pos2 == r
        l = jnp.min(jnp.where(rsel, FL, BIGI))
        row = pv_ref[pl.ds(r, 1), :]
        outv = jnp.where(iota384 == t, gm, outv)
        outn = jnp.where(iota384 == t, r, outn)
        outl = jnp.where(iota384 == t, l, outl)
        newrow = jnp.where(iota_l == l, NEG, row)
        pv_ref[pl.ds(r, 1), :] = newrow
        nv = jnp.max(newrow)
        nfl = jnp.min(jnp.where(newrow == nv, iota_l, BIGI))
        vmask = jnp.where(rsel, NEG, V)
        m2 = jnp.max(vmask)
        rA = jnp.min(jnp.where(vmask == m2, pos2, BIGI))
        gm_n = jnp.maximum(nv, m2)
        r_n = jnp.where(nv > m2, r, jnp.where(nv < m2, rA, jnp.minimum(r, rA)))
        V = jnp.where(rsel, nv, V)
        FL = jnp.where(rsel, nfl, FL)
        return V, FL, gm_n, r_n, outv, outn, outl

    outv0 = jnp.full((1, 384), NEG, jnp.float32)
    outi0 = jnp.zeros((1, 384), jnp.int32)
    gm0 = jnp.max(V)
    r0 = jnp.min(jnp.where(V == gm0, pos2, BIGI))
    _, _, _, _, outv, outn, outl = jax.lax.fori_loop(
        0, K_SELECT, body, (V, FL, gm0, r0, outv0, outi0, outi0))

    scores_ref[0, 0] = outv[0, :K_SELECT]
    labels_ref[0, 0] = outl[0, :K_SELECT]
    n_idx = outn[0, :K_SELECT]
    nidx_ref[0, 0] = n_idx

    bx = boxes_ref[0]
    cx, cy, w, h = bx[:, 0:1], bx[:, 1:2], bx[:, 2:3], bx[:, 3:4]
    xyxy = jnp.concatenate(
        [cx - 0.5 * w, cy - 0.5 * h, cx + 0.5 * w, cy + 0.5 * h], axis=-1)
    iota_n = jax.lax.broadcasted_iota(jnp.int32, (K_SELECT, N_ROWS), 1)
    oh = jnp.where(n_idx[:, None] == iota_n, 1.0, 0.0)
    sel = jnp.dot(oh, xyxy, preferred_element_type=jnp.float32,
                  precision=jax.lax.Precision.HIGHEST)
    boxes_out_ref[0, 0] = sel * scale_ref[0, 0]


def _fused(pred_logits, wt_pad, pred_boxes, scale):
    B = pred_logits.shape[0]
    return pl.pallas_call(
        _fused_body,
        grid=(B,),
        in_specs=[
            pl.BlockSpec((1, N_ROWS, 256), lambda b: (b, 0, 0)),
            pl.BlockSpec((256, L_PAD), lambda b: (0, 0)),
            pl.BlockSpec((1, N_ROWS, 4), lambda b: (b, 0, 0)),
            pl.BlockSpec((1, 1, 4), lambda b: (b, 0, 0)),
        ],
        out_specs=[
            pl.BlockSpec((1, 1, K_SELECT), lambda b: (b, 0, 0)),
            pl.BlockSpec((1, 1, K_SELECT), lambda b: (b, 0, 0)),
            pl.BlockSpec((1, 1, K_SELECT), lambda b: (b, 0, 0)),
            pl.BlockSpec((1, 1, K_SELECT, 4), lambda b: (b, 0, 0, 0)),
        ],
        out_shape=[
            jax.ShapeDtypeStruct((B, 1, K_SELECT), jnp.float32),
            jax.ShapeDtypeStruct((B, 1, K_SELECT), jnp.int32),
            jax.ShapeDtypeStruct((B, 1, K_SELECT), jnp.int32),
            jax.ShapeDtypeStruct((B, 1, K_SELECT, 4), jnp.float32),
        ],
        scratch_shapes=[
            pltpu.VMEM((N_ROWS, L_PAD), jnp.float32),
        ],
    )(pred_logits, wt_pad, pred_boxes, scale)


def kernel(pred_logits, pred_boxes, target_sizes, image_names, label_positive_map):
    B = pred_logits.shape[0]
    wt_pad = jnp.zeros((256, L_PAD), jnp.float32).at[:, :L_REAL].set(
        label_positive_map.T)
    ts = target_sizes.astype(jnp.float32)
    scale = jnp.stack([ts[:, 1], ts[:, 0], ts[:, 1], ts[:, 0]],
                      axis=1).reshape(B, 1, 4)

    scores, labels, n_idx, boxes = _fused(pred_logits, wt_pad, pred_boxes, scale)
    scores = scores.reshape(B, K_SELECT)
    labels = labels.reshape(B, K_SELECT)
    n_idx = n_idx.reshape(B, K_SELECT)
    boxes = boxes.reshape(B, K_SELECT, 4)

    text_masks = jax.nn.sigmoid(pred_logits) > 0.0
    idxV = jnp.broadcast_to(n_idx[:, :, None], (B, K_SELECT, 256))
    text_masks = jnp.take_along_axis(text_masks, idxV, axis=1)

    return (text_masks, scores, labels, boxes)


# speedup vs baseline: 1.3129x; 1.3129x over previous
import jax
import jax.numpy as jnp
from jax.experimental import pallas as pl
from jax.experimental.pallas import tpu as pltpu

K_SELECT = 300
L_REAL = 1203
L_PAD = 1280
N_ROWS = 5000
NTILE = 5
TN = N_ROWS // NTILE
NEG = -1e30
BIGI = 2_000_000_000


def _fused_body(logits_ref, wt_ref, boxes_ref, scale_ref,
                scores_ref, labels_ref, nidx_ref, boxes_out_ref, pv_ref):
    col = jax.lax.broadcasted_iota(jnp.int32, (TN, L_PAD), 1)
    v_parts = []
    for t in range(NTILE):
        x = logits_ref[0, t * TN:(t + 1) * TN, :]
        s = jax.nn.sigmoid(x)
        probs = jnp.dot(s, wt_ref[...], preferred_element_type=jnp.float32)
        probs = jnp.where(col < L_REAL, probs, NEG)
        pv_ref[t * TN:(t + 1) * TN, :] = probs
        v_parts.append(jnp.max(probs, axis=1))
    V = jnp.stack(v_parts)

    pos2 = jax.lax.broadcasted_iota(jnp.int32, (NTILE, TN), 0) * TN + \
        jax.lax.broadcasted_iota(jnp.int32, (NTILE, TN), 1)
    iota_l = jax.lax.broadcasted_iota(jnp.int32, (1, L_PAD), 1)
    iota384 = jax.lax.broadcasted_iota(jnp.int32, (1, 384), 1)

    def body(t, carry):
        V, gm, r, outv, outn, outl = carry
        row = pv_ref[pl.ds(r, 1), :]
        l = jnp.min(jnp.where(row == gm, iota_l, BIGI))
        outv = jnp.where(iota384 == t, gm, outv)
        outn = jnp.where(iota384 == t, r, outn)
        outl = jnp.where(iota384 == t, l, outl)
        newrow = jnp.where(iota_l == l, NEG, row)
        pv_ref[pl.ds(r, 1), :] = newrow
        nv = jnp.max(newrow)
        vmask = jnp.where(pos2 == r, NEG, V)
        m2 = jnp.max(vmask)
        rA = jnp.min(jnp.where(vmask == m2, pos2, BIGI))
        gm_n = jnp.maximum(nv, m2)
        r_n = jnp.where(nv > m2, r, jnp.where(nv < m2, rA, jnp.minimum(r, rA)))
        V = jnp.where(pos2 == r, nv, V)
        return V, gm_n, r_n, outv, outn, outl

    outv0 = jnp.full((1, 384), NEG, jnp.float32)
    outi0 = jnp.zeros((1, 384), jnp.int32)
    gm0 = jnp.max(V)
    r0 = jnp.min(jnp.where(V == gm0, pos2, BIGI))
    _, _, _, outv, outn, outl = jax.lax.fori_loop(
        0, K_SELECT, body, (V, gm0, r0, outv0, outi0, outi0))

    scores_ref[0, 0] = outv[0, :K_SELECT]
    labels_ref[0, 0] = outl[0, :K_SELECT]
    n_idx = outn[0, :K_SELECT]
    nidx_ref[0, 0] = n_idx

    bx = boxes_ref[0]
    cx, cy, w, h = bx[:, 0:1], bx[:, 1:2], bx[:, 2:3], bx[:, 3:4]
    xyxy = jnp.concatenate(
        [cx - 0.5 * w, cy - 0.5 * h, cx + 0.5 * w, cy + 0.5 * h], axis=-1)
    iota_n = jax.lax.broadcasted_iota(jnp.int32, (K_SELECT, N_ROWS), 1)
    oh = jnp.where(n_idx[:, None] == iota_n, 1.0, 0.0)
    sel = jnp.dot(oh, xyxy, preferred_element_type=jnp.float32,
                  precision=jax.lax.Precision.HIGHEST)
    boxes_out_ref[0, 0] = sel * scale_ref[0, 0]


def _fused(pred_logits, wt_pad, pred_boxes, scale):
    B = pred_logits.shape[0]
    return pl.pallas_call(
        _fused_body,
        grid=(B,),
        in_specs=[
            pl.BlockSpec((1, N_ROWS, 256), lambda b: (b, 0, 0)),
            pl.BlockSpec((256, L_PAD), lambda b: (0, 0)),
            pl.BlockSpec((1, N_ROWS, 4), lambda b: (b, 0, 0)),
            pl.BlockSpec((1, 1, 4), lambda b: (b, 0, 0)),
        ],
        out_specs=[
            pl.BlockSpec((1, 1, K_SELECT), lambda b: (b, 0, 0)),
            pl.BlockSpec((1, 1, K_SELECT), lambda b: (b, 0, 0)),
            pl.BlockSpec((1, 1, K_SELECT), lambda b: (b, 0, 0)),
            pl.BlockSpec((1, 1, K_SELECT, 4), lambda b: (b, 0, 0, 0)),
        ],
        out_shape=[
            jax.ShapeDtypeStruct((B, 1, K_SELECT), jnp.float32),
            jax.ShapeDtypeStruct((B, 1, K_SELECT), jnp.int32),
            jax.ShapeDtypeStruct((B, 1, K_SELECT), jnp.int32),
            jax.ShapeDtypeStruct((B, 1, K_SELECT, 4), jnp.float32),
        ],
        scratch_shapes=[
            pltpu.VMEM((N_ROWS, L_PAD), jnp.float32),
        ],
    )(pred_logits, wt_pad, pred_boxes, scale)


def kernel(pred_logits, pred_boxes, target_sizes, image_names, label_positive_map):
    B = pred_logits.shape[0]
    wt_pad = jnp.zeros((256, L_PAD), jnp.float32).at[:, :L_REAL].set(
        label_positive_map.T)
    ts = target_sizes.astype(jnp.float32)
    scale = jnp.stack([ts[:, 1], ts[:, 0], ts[:, 1], ts[:, 0]],
                      axis=1).reshape(B, 1, 4)

    scores, labels, n_idx, boxes = _fused(pred_logits, wt_pad, pred_boxes, scale)
    scores = scores.reshape(B, K_SELECT)
    labels = labels.reshape(B, K_SELECT)
    n_idx = n_idx.reshape(B, K_SELECT)
    boxes = boxes.reshape(B, K_SELECT, 4)

    text_masks = jax.nn.sigmoid(pred_logits) > 0.0
    idxV = jnp.broadcast_to(n_idx[:, :, None], (B, K_SELECT, 256))
    text_masks = jnp.take_along_axis(text_masks, idxV, axis=1)

    return (text_masks, scores, labels, boxes)


# speedup vs baseline: 1.3302x; 1.0131x over previous
import jax
import jax.numpy as jnp
from jax.experimental import pallas as pl
from jax.experimental.pallas import tpu as pltpu

K_SELECT = 300
L_REAL = 1203
L_PAD = 1280
N_ROWS = 5000
NTILE = 5
TN = N_ROWS // NTILE
NEG = -1e30
BIGI = 2_000_000_000


def _fused_body(logits_ref, wt_ref, boxes_ref, scale_ref,
                scores_ref, labels_ref, nidx_ref, boxes_out_ref, pv_ref):
    col = jax.lax.broadcasted_iota(jnp.int32, (TN, L_PAD), 1)
    v_parts = []
    for t in range(NTILE):
        x = logits_ref[0, t * TN:(t + 1) * TN, :]
        s = jax.nn.sigmoid(x)
        probs = jnp.dot(s, wt_ref[...], preferred_element_type=jnp.float32)
        probs = jnp.where(col < L_REAL, probs, NEG)
        pv_ref[t * TN:(t + 1) * TN, :] = probs
        v_parts.append(jnp.max(probs, axis=1))
    V = jnp.stack(v_parts)

    pos2 = jax.lax.broadcasted_iota(jnp.int32, (NTILE, TN), 0) * TN + \
        jax.lax.broadcasted_iota(jnp.int32, (NTILE, TN), 1)
    iota_l = jax.lax.broadcasted_iota(jnp.int32, (1, L_PAD), 1)
    iota384 = jax.lax.broadcasted_iota(jnp.int32, (1, 384), 1)

    def body(t, carry):
        V, gm, r, outv, outn, outl = carry
        row = pv_ref[pl.ds(r, 1), :]
        l = jnp.min(jnp.where(row == gm, iota_l, BIGI))
        outv = jnp.where(iota384 == t, gm, outv)
        outn = jnp.where(iota384 == t, r, outn)
        outl = jnp.where(iota384 == t, l, outl)
        newrow = jnp.where(iota_l == l, NEG, row)
        pv_ref[pl.ds(r, 1), :] = newrow
        nv = jnp.max(newrow)
        vmask = jnp.where(pos2 == r, NEG, V)
        m2 = jnp.max(vmask)
        rA = jnp.min(jnp.where(vmask == m2, pos2, BIGI))
        gm_n = jnp.maximum(nv, m2)
        r_n = jnp.where(nv > m2, r, jnp.where(nv < m2, rA, jnp.minimum(r, rA)))
        V = jnp.where(pos2 == r, nv, V)
        return V, gm_n, r_n, outv, outn, outl

    outv0 = jnp.full((1, 384), NEG, jnp.float32)
    outi0 = jnp.zeros((1, 384), jnp.int32)
    gm0 = jnp.max(V)
    r0 = jnp.min(jnp.where(V == gm0, pos2, BIGI))
    _, _, _, outv, outn, outl = jax.lax.fori_loop(
        0, K_SELECT, body, (V, gm0, r0, outv0, outi0, outi0))

    scores_ref[0, 0] = outv[0, :K_SELECT]
    labels_ref[0, 0] = outl[0, :K_SELECT]
    n_idx = outn[0, :K_SELECT]
    nidx_ref[0, 0] = n_idx

    bx = boxes_ref[0]
    cx, cy, w, h = bx[:, 0:1], bx[:, 1:2], bx[:, 2:3], bx[:, 3:4]
    xyxy = jnp.concatenate(
        [cx - 0.5 * w, cy - 0.5 * h, cx + 0.5 * w, cy + 0.5 * h], axis=-1)
    iota_n = jax.lax.broadcasted_iota(jnp.int32, (K_SELECT, N_ROWS), 1)
    oh = jnp.where(n_idx[:, None] == iota_n, 1.0, 0.0)
    sel = jnp.dot(oh, xyxy, preferred_element_type=jnp.float32,
                  precision=jax.lax.Precision.HIGHEST)
    boxes_out_ref[0, 0] = sel * scale_ref[0, 0]


def _fused(pred_logits, wt_pad, pred_boxes, scale):
    B = pred_logits.shape[0]
    return pl.pallas_call(
        _fused_body,
        grid=(B,),
        in_specs=[
            pl.BlockSpec((1, N_ROWS, 256), lambda b: (b, 0, 0)),
            pl.BlockSpec((256, L_PAD), lambda b: (0, 0)),
            pl.BlockSpec((1, N_ROWS, 4), lambda b: (b, 0, 0)),
            pl.BlockSpec((1, 1, 4), lambda b: (b, 0, 0)),
        ],
        out_specs=[
            pl.BlockSpec((1, 1, K_SELECT), lambda b: (b, 0, 0)),
            pl.BlockSpec((1, 1, K_SELECT), lambda b: (b, 0, 0)),
            pl.BlockSpec((1, 1, K_SELECT), lambda b: (b, 0, 0)),
            pl.BlockSpec((1, 1, K_SELECT, 4), lambda b: (b, 0, 0, 0)),
        ],
        out_shape=[
            jax.ShapeDtypeStruct((B, 1, K_SELECT), jnp.float32),
            jax.ShapeDtypeStruct((B, 1, K_SELECT), jnp.int32),
            jax.ShapeDtypeStruct((B, 1, K_SELECT), jnp.int32),
            jax.ShapeDtypeStruct((B, 1, K_SELECT, 4), jnp.float32),
        ],
        scratch_shapes=[
            pltpu.VMEM((N_ROWS, L_PAD), jnp.float32),
        ],
    )(pred_logits, wt_pad, pred_boxes, scale)


def kernel(pred_logits, pred_boxes, target_sizes, image_names, label_positive_map):
    B = pred_logits.shape[0]
    wt_pad = jnp.zeros((256, L_PAD), jnp.float32).at[:, :L_REAL].set(
        label_positive_map.T)
    ts = target_sizes.astype(jnp.float32)
    scale = jnp.stack([ts[:, 1], ts[:, 0], ts[:, 1], ts[:, 0]],
                      axis=1).reshape(B, 1, 4)

    scores, labels, n_idx, boxes = _fused(pred_logits, wt_pad, pred_boxes, scale)
    scores = scores.reshape(B, K_SELECT)
    labels = labels.reshape(B, K_SELECT)
    n_idx = n_idx.reshape(B, K_SELECT)
    boxes = boxes.reshape(B, K_SELECT, 4)

    idxV = jnp.broadcast_to(n_idx[:, :, None], (B, K_SELECT, 256))
    sel_logits = jnp.take_along_axis(pred_logits, idxV, axis=1)
    text_masks = jax.nn.sigmoid(sel_logits) > 0.0

    return (text_masks, scores, labels, boxes)
